# Initial kernel scaffold; baseline (speedup 1.0000x reference)
import jax
import jax.numpy as jnp
from jax.experimental import pallas as pl


def kernel(u_pos, v_pos, v_neg, batch_size, u_weight, v_weight):
    raise NotImplementedError("write your pallas kernel here")


# speedup vs baseline: 2.5903x; 2.5903x over previous
import jax
import jax.numpy as jnp
from jax import lax
from jax.experimental import pallas as pl
from jax.experimental.pallas import tpu as pltpu
from jax.experimental.pallas import tpu_sc as plsc

NC = 2
NS = 16
L = 16
NW = NC * NS

V = 1000000
D = 64
B = 16384
NNEG = 5
NIDX = 2 + NNEG

CPW = B // NW
CHUNK = 128
NCHUNK = CPW // CHUNK


def _sc_body(u_w, v_w, up, vp, vn_flat, pos_out, neg_out, *scratch):
    idx = scratch[0:NIDX]
    u_rows, v_rows = scratch[NIDX], scratch[NIDX + 1]
    n_rows = scratch[NIDX + 2:NIDX + 2 + NNEG]
    pos_buf, neg_buf, sem = scratch[NIDX + 2 + NNEG:]

    wid = lax.axis_index("s") * NC + lax.axis_index("c")
    base = wid * CPW

    srcs = [up.at[pl.ds(base, CPW)], vp.at[pl.ds(base, CPW)]] + [
        vn_flat.at[pl.ds(n * B + base, CPW)] for n in range(NNEG)]
    for k in range(NIDX):
        pltpu.sync_copy(srcs[k], idx[k].at[pl.ds(0, CPW)])

    tabs = [u_w, v_w] + [v_w] * NNEG
    bufs = [u_rows, v_rows] + list(n_rows)

    def issue(ck):
        cbase = ck * CHUNK

        def issue_body(b, _):
            gb = cbase + b
            for k in range(NIDX):
                r = idx[k][pl.ds(gb, L)][0]
                pltpu.async_copy(tabs[k].at[pl.ds(r, 1)],
                                 bufs[k].at[pl.ds(b, 1)], sem)
            return 0

        lax.fori_loop(0, CHUNK, issue_body, 0)

    def drain():
        for k in range(NIDX):
            pltpu.make_async_copy(tabs[k].at[pl.ds(0, CHUNK)], bufs[k],
                                  sem).wait()

    issue(0)
    for ck in range(NCHUNK):
        drain()
        if ck + 1 < NCHUNK:
            issue(ck + 1)
        cbase = ck * CHUNK

        last_lane = lax.iota(jnp.int32, L) == (L - 1)

        def elem_body(b, _, cbase=cbase):
            pos_part = None
            neg_part = None
            for j in range(D // L):
                sl = pl.ds(j * L, L)
                vu = u_rows[b, sl]
                vv = v_rows[b, sl]
                vns = None
                for n in range(NNEG):
                    vn = n_rows[n][b, sl]
                    vns = vn if vns is None else vns + vn
                pp = vu * vv
                np_ = vu * vns
                pos_part = pp if pos_part is None else pos_part + pp
                neg_part = np_ if neg_part is None else neg_part + np_
            out_idx = jnp.full((L,), cbase + b, jnp.int32)
            plsc.store_scatter(pos_buf, [out_idx], plsc.cumsum(pos_part),
                               mask=last_lane)
            plsc.store_scatter(neg_buf, [out_idx], plsc.cumsum(neg_part),
                               mask=last_lane)
            return 0

        lax.fori_loop(0, CHUNK, elem_body, 0)

    pltpu.sync_copy(pos_buf, pos_out.at[pl.ds(base, CPW)])
    pltpu.sync_copy(neg_buf, neg_out.at[pl.ds(base, CPW)])


def _sc_scores(u_pos, v_pos, v_neg_flat, u_weight, v_weight):
    mesh = plsc.VectorSubcoreMesh(
        core_axis_name="c", subcore_axis_name="s", num_cores=NC, num_subcores=NS)
    scratch = (
        [pltpu.VMEM((CPW + L,), jnp.int32) for _ in range(NIDX)]
        + [pltpu.VMEM((CHUNK, D), jnp.float32) for _ in range(2 + NNEG)]
        + [pltpu.VMEM((CPW,), jnp.float32)] * 2
        + [pltpu.SemaphoreType.DMA]
    )
    f = pl.kernel(
        _sc_body,
        out_type=(
            jax.ShapeDtypeStruct((B,), jnp.float32),
            jax.ShapeDtypeStruct((B,), jnp.float32),
        ),
        mesh=mesh,
        scratch_types=scratch,
        compiler_params=pltpu.CompilerParams(
            needs_layout_passes=False, use_tc_tiling_on_sc=True),
    )
    return f(u_weight, v_weight, u_pos, v_pos, v_neg_flat)


def _loss_body(pos_ref, neg_ref, o_ref):
    x = pos_ref[...]
    y = neg_ref[...]

    def log_sigmoid(t):
        return jnp.minimum(t, 0.0) - jnp.log1p(jnp.exp(-jnp.abs(t)))

    o_ref[0, 0] = jnp.sum(log_sigmoid(x) + log_sigmoid(-y))


def _tc_loss(pos_s, neg_s):
    f = pl.pallas_call(
        _loss_body,
        out_shape=jax.ShapeDtypeStruct((1, 1), jnp.float32),
        out_specs=pl.BlockSpec(memory_space=pltpu.SMEM),
    )
    return f(pos_s.reshape(128, B // 128), neg_s.reshape(128, B // 128))


def kernel(u_pos, v_pos, v_neg, batch_size, u_weight, v_weight):
    v_neg_flat = v_neg.T.reshape(-1)
    pos_s, neg_s = _sc_scores(u_pos, v_pos, v_neg_flat, u_weight, v_weight)
    total = _tc_loss(pos_s, neg_s)
    return -1.0 * total[0, 0] / batch_size


# speedup vs baseline: 3.0501x; 1.1775x over previous
import jax
import jax.numpy as jnp
from jax import lax
from jax.experimental import pallas as pl
from jax.experimental.pallas import tpu as pltpu
from jax.experimental.pallas import tpu_sc as plsc

NC = 2
NS = 16
L = 16
NW = NC * NS

V = 1000000
D = 64
B = 16384
NNEG = 5
NIDX = 2 + NNEG

W = 1664
G = 301
H = G * W

CPW = B // NW
CHUNK = 128
NCHUNK = CPW // CHUNK


def _sc_body(u_w, v_w, up, vp, vn_flat, pos_out, neg_out, *scratch):
    idx = scratch[0:NIDX]
    u_rows, v_rows = scratch[NIDX], scratch[NIDX + 1]
    n_rows = scratch[NIDX + 2:NIDX + 2 + NNEG]
    pos_buf, neg_buf, sem = scratch[NIDX + 2 + NNEG:]

    wid = lax.axis_index("s") * NC + lax.axis_index("c")
    base = wid * CPW

    srcs = [up.at[pl.ds(base, CPW)], vp.at[pl.ds(base, CPW)]] + [
        vn_flat.at[pl.ds(n * B + base, CPW)] for n in range(NNEG)]
    for k in range(NIDX):
        pltpu.sync_copy(srcs[k], idx[k].at[pl.ds(0, CPW)])

    tabs = [u_w, v_w] + [v_w] * NNEG
    bufs = [u_rows, v_rows] + list(n_rows)

    def issue(ck):
        cbase = ck * CHUNK

        def issue_body(b, _):
            gb = cbase + b
            for k in range(NIDX):
                r = idx[k][pl.ds(gb, L)][0]
                q = r - jnp.where(r >= H, H, 0)
                pltpu.async_copy(tabs[k].at[pl.ds(q, 1)],
                                 bufs[k].at[pl.ds(b, 1)], sem)
            return 0

        lax.fori_loop(0, CHUNK, issue_body, 0)

    def drain():
        for k in range(NIDX):
            pltpu.make_async_copy(tabs[k].at[pl.ds(0, CHUNK)], bufs[k],
                                  sem).wait()

    issue(0)
    for ck in range(NCHUNK):
        drain()
        if ck + 1 < NCHUNK:
            issue(ck + 1)
        cbase = ck * CHUNK

        last_lane = lax.iota(jnp.int32, L) == (L - 1)

        def elem_body(b, _, cbase=cbase):
            gb = cbase + b
            offs = [jnp.where(idx[k][pl.ds(gb, L)][0] >= H, D, 0)
                    for k in range(NIDX)]
            pos_part = None
            neg_part = None
            for j in range(D // L):
                vu = u_rows[b, pl.ds(offs[0] + j * L, L)]
                vv = v_rows[b, pl.ds(offs[1] + j * L, L)]
                vns = None
                for n in range(NNEG):
                    vn = n_rows[n][b, pl.ds(offs[2 + n] + j * L, L)]
                    vns = vn if vns is None else vns + vn
                pp = vu * vv
                np_ = vu * vns
                pos_part = pp if pos_part is None else pos_part + pp
                neg_part = np_ if neg_part is None else neg_part + np_
            out_idx = jnp.full((L,), gb, jnp.int32)
            plsc.store_scatter(pos_buf, [out_idx], plsc.cumsum(pos_part),
                               mask=last_lane)
            plsc.store_scatter(neg_buf, [out_idx], plsc.cumsum(neg_part),
                               mask=last_lane)
            return 0

        lax.fori_loop(0, CHUNK, elem_body, 0)

    pltpu.sync_copy(pos_buf, pos_out.at[pl.ds(base, CPW)])
    pltpu.sync_copy(neg_buf, neg_out.at[pl.ds(base, CPW)])


def _sc_scores(u_pos, v_pos, v_neg_flat, u_w2, v_w2):
    mesh = plsc.VectorSubcoreMesh(
        core_axis_name="c", subcore_axis_name="s", num_cores=NC, num_subcores=NS)
    scratch = (
        [pltpu.VMEM((CPW + L,), jnp.int32) for _ in range(NIDX)]
        + [pltpu.VMEM((CHUNK, 2 * D), jnp.float32) for _ in range(2 + NNEG)]
        + [pltpu.VMEM((CPW,), jnp.float32)] * 2
        + [pltpu.SemaphoreType.DMA]
    )
    f = pl.kernel(
        _sc_body,
        out_type=(
            jax.ShapeDtypeStruct((B,), jnp.float32),
            jax.ShapeDtypeStruct((B,), jnp.float32),
        ),
        mesh=mesh,
        scratch_types=scratch,
        compiler_params=pltpu.CompilerParams(
            needs_layout_passes=False, use_tc_tiling_on_sc=True),
    )
    return f(u_w2, v_w2, u_pos, v_pos, v_neg_flat)


def _stage_body(ua_ref, ub_ref, va_ref, vb_ref, ou_ref, ov_ref):
    for j in range(W // 128):
        sl = pl.ds(j * 128, 128)
        ou_ref[sl, :] = jnp.concatenate(
            [ua_ref[:, sl].T, ub_ref[:, sl].T], axis=1)
        ov_ref[sl, :] = jnp.concatenate(
            [va_ref[:, sl].T, vb_ref[:, sl].T], axis=1)


def _stage(u_t, v_t):
    left = pl.BlockSpec((D, W), lambda c: (0, c))
    right = pl.BlockSpec((D, W), lambda c: (0, jnp.minimum(c + G, 2 * G - 2)))
    out = pl.BlockSpec((W, 2 * D), lambda c: (c, 0))
    f = pl.pallas_call(
        _stage_body,
        grid=(G,),
        in_specs=[left, right, left, right],
        out_specs=[out, out],
        out_shape=[jax.ShapeDtypeStruct((H, 2 * D), jnp.float32)] * 2,
    )
    return f(u_t, u_t, v_t, v_t)


def _loss_body(pos_ref, neg_ref, o_ref):
    x = pos_ref[...]
    y = neg_ref[...]

    def log_sigmoid(t):
        return jnp.minimum(t, 0.0) - jnp.log1p(jnp.exp(-jnp.abs(t)))

    o_ref[0, 0] = jnp.sum(log_sigmoid(x) + log_sigmoid(-y))


def _tc_loss(pos_s, neg_s):
    f = pl.pallas_call(
        _loss_body,
        out_shape=jax.ShapeDtypeStruct((1, 1), jnp.float32),
        out_specs=pl.BlockSpec(memory_space=pltpu.SMEM),
    )
    return f(pos_s.reshape(128, B // 128), neg_s.reshape(128, B // 128))


def kernel(u_pos, v_pos, v_neg, batch_size, u_weight, v_weight):
    v_neg_flat = v_neg.T.reshape(-1)
    u_w2, v_w2 = _stage(u_weight.T, v_weight.T)
    pos_s, neg_s = _sc_scores(u_pos, v_pos, v_neg_flat, u_w2, v_w2)
    total = _tc_loss(pos_s, neg_s)
    return -1.0 * total[0, 0] / batch_size


# speedup vs baseline: 3.4921x; 1.1449x over previous
import jax
import jax.numpy as jnp
from jax import lax
from jax.experimental import pallas as pl
from jax.experimental.pallas import tpu as pltpu
from jax.experimental.pallas import tpu_sc as plsc

NC = 2
NS = 16
L = 16
NW = NC * NS

V = 1000000
D = 64
B = 16384
NNEG = 5
NIDX = 2 + NNEG

W = 1664
G = 302
H = G * W
GS = G // 2

CPW = B // NW
CHUNK = 128
NCHUNK = CPW // CHUNK


def _sc_body(u_w, v_w, up, vp, vn_flat, pos_out, neg_out, *scratch):
    idx = scratch[0:NIDX]
    u_rows, v_rows = scratch[NIDX], scratch[NIDX + 1]
    n_rows = scratch[NIDX + 2:NIDX + 2 + NNEG]
    pos_buf, neg_buf, sem = scratch[NIDX + 2 + NNEG:]

    wid = lax.axis_index("s") * NC + lax.axis_index("c")
    base = wid * CPW

    srcs = [up.at[pl.ds(base, CPW)], vp.at[pl.ds(base, CPW)]] + [
        vn_flat.at[pl.ds(n * B + base, CPW)] for n in range(NNEG)]
    for k in range(NIDX):
        pltpu.sync_copy(srcs[k], idx[k].at[pl.ds(0, CPW)])

    tabs = [u_w, v_w] + [v_w] * NNEG
    bufs = [u_rows, v_rows] + list(n_rows)

    def issue(ck):
        cbase = ck * CHUNK

        def issue_body(b, _):
            gb = cbase + b
            for k in range(NIDX):
                r = idx[k][pl.ds(gb, L)][0]
                q = r - jnp.where(r >= H, H, 0)
                pltpu.async_copy(tabs[k].at[pl.ds(q, 1)],
                                 bufs[k].at[pl.ds(b, 1)], sem)
            return 0

        lax.fori_loop(0, CHUNK, issue_body, 0)

    def drain():
        for k in range(NIDX):
            pltpu.make_async_copy(tabs[k].at[pl.ds(0, CHUNK)], bufs[k],
                                  sem).wait()

    issue(0)
    for ck in range(NCHUNK):
        drain()
        if ck + 1 < NCHUNK:
            issue(ck + 1)
        cbase = ck * CHUNK

        last_lane = lax.iota(jnp.int32, L) == (L - 1)

        def elem_body(b, _, cbase=cbase):
            gb = cbase + b
            offs = [jnp.where(idx[k][pl.ds(gb, L)][0] >= H, D, 0)
                    for k in range(NIDX)]
            pos_part = None
            neg_part = None
            for j in range(D // L):
                vu = u_rows[b, pl.ds(offs[0] + j * L, L)]
                vv = v_rows[b, pl.ds(offs[1] + j * L, L)]
                vns = None
                for n in range(NNEG):
                    vn = n_rows[n][b, pl.ds(offs[2 + n] + j * L, L)]
                    vns = vn if vns is None else vns + vn
                pp = vu * vv
                np_ = vu * vns
                pos_part = pp if pos_part is None else pos_part + pp
                neg_part = np_ if neg_part is None else neg_part + np_
            out_idx = jnp.full((L,), gb, jnp.int32)
            plsc.store_scatter(pos_buf, [out_idx], plsc.cumsum(pos_part),
                               mask=last_lane)
            plsc.store_scatter(neg_buf, [out_idx], plsc.cumsum(neg_part),
                               mask=last_lane)
            return 0

        lax.fori_loop(0, CHUNK, elem_body, 0)

    pltpu.sync_copy(pos_buf, pos_out.at[pl.ds(base, CPW)])
    pltpu.sync_copy(neg_buf, neg_out.at[pl.ds(base, CPW)])


def _sc_scores(u_pos, v_pos, v_neg_flat, u_w2, v_w2):
    mesh = plsc.VectorSubcoreMesh(
        core_axis_name="c", subcore_axis_name="s", num_cores=NC, num_subcores=NS)
    scratch = (
        [pltpu.VMEM((CPW + L,), jnp.int32) for _ in range(NIDX)]
        + [pltpu.VMEM((CHUNK, 2 * D), jnp.float32) for _ in range(2 + NNEG)]
        + [pltpu.VMEM((CPW,), jnp.float32)] * 2
        + [pltpu.SemaphoreType.DMA]
    )
    f = pl.kernel(
        _sc_body,
        out_type=(
            jax.ShapeDtypeStruct((B,), jnp.float32),
            jax.ShapeDtypeStruct((B,), jnp.float32),
        ),
        mesh=mesh,
        scratch_types=scratch,
        compiler_params=pltpu.CompilerParams(
            needs_layout_passes=False, use_tc_tiling_on_sc=True),
    )
    return f(u_w2, v_w2, u_pos, v_pos, v_neg_flat)


def _stage_body(ua0, ua1, ub0, ub1, va0, va1, vb0, vb1, ou_ref, ov_ref):
    for h, (ua, ub, va, vb) in enumerate(((ua0, ub0, va0, vb0),
                                          (ua1, ub1, va1, vb1))):
        for j in range(W // 128):
            sl = pl.ds(j * 128, 128)
            osl = pl.ds(h * W + j * 128, 128)
            ou_ref[osl, :] = jnp.concatenate(
                [ua[:, sl].T, ub[:, sl].T], axis=1)
            ov_ref[osl, :] = jnp.concatenate(
                [va[:, sl].T, vb[:, sl].T], axis=1)


def _stage(u_t, v_t):
    last = (V + W - 1) // W
    left0 = pl.BlockSpec((D, W), lambda c: (0, 2 * c))
    left1 = pl.BlockSpec((D, W), lambda c: (0, 2 * c + 1))
    right0 = pl.BlockSpec(
        (D, W), lambda c: (0, jnp.minimum(2 * c + G, last - 1)))
    right1 = pl.BlockSpec(
        (D, W), lambda c: (0, jnp.minimum(2 * c + 1 + G, last - 1)))
    out = pl.BlockSpec((2 * W, 2 * D), lambda c: (c, 0))
    f = pl.pallas_call(
        _stage_body,
        grid=(GS,),
        in_specs=[left0, left1, right0, right1] * 2,
        out_specs=[out, out],
        out_shape=[jax.ShapeDtypeStruct((H, 2 * D), jnp.float32)] * 2,
    )
    return f(u_t, u_t, u_t, u_t, v_t, v_t, v_t, v_t)


def _loss_body(pos_ref, neg_ref, o_ref):
    x = pos_ref[...]
    y = neg_ref[...]

    def log_sigmoid(t):
        return jnp.minimum(t, 0.0) - jnp.log1p(jnp.exp(-jnp.abs(t)))

    o_ref[0, 0] = jnp.sum(log_sigmoid(x) + log_sigmoid(-y))


def _tc_loss(pos_s, neg_s):
    f = pl.pallas_call(
        _loss_body,
        out_shape=jax.ShapeDtypeStruct((1, 1), jnp.float32),
        out_specs=pl.BlockSpec(memory_space=pltpu.SMEM),
    )
    return f(pos_s.reshape(128, B // 128), neg_s.reshape(128, B // 128))


def kernel(u_pos, v_pos, v_neg, batch_size, u_weight, v_weight):
    v_neg_flat = v_neg.T.reshape(-1)
    u_w2, v_w2 = _stage(u_weight.T, v_weight.T)
    pos_s, neg_s = _sc_scores(u_pos, v_pos, v_neg_flat, u_w2, v_w2)
    total = _tc_loss(pos_s, neg_s)
    return -1.0 * total[0, 0] / batch_size
